# baseline (speedup 1.0000x reference)
import jax
import jax.numpy as jnp
from jax import lax
from jax.experimental import pallas as pl
from jax.experimental.pallas import tpu as pltpu
from jax.experimental.pallas import tpu_sc as plsc

_MAXLEN = 200
_EMBED = 64
_VOCAB = 100000
_BATCH = 4096
_LANES = 16
_PADDED = 128

_NC = 2
_NS = 16
_NW = _NC * _NS
_BLK = _BATCH // _NW
_ET = _EMBED // 8
_TILE = 8 * _BLK
_ROUNDS = _MAXLEN // 2
_J = _EMBED // _LANES


def _tec_body(x_hbm, tok_hbm, pos_hbm, out_hbm, pos_v, idx_v, rows_v, slab_v,
              g0, g1, o0, o1):
    c = lax.axis_index("c")
    s = lax.axis_index("s")
    wid = s * _NC + c
    gsems = (g0, g1)
    osems = (o0, o1)
    pltpu.sync_copy(pos_hbm, pos_v)
    b0 = wid * _BLK
    lane = jax.lax.iota(jnp.int32, _LANES)
    et_hi = jax.lax.shift_right_logical(lane, 3)
    in_lo = jax.lax.shift_left(jnp.bitwise_and(lane, 7), 7)
    et_idx = [et_hi + (k * _LANES) // 8 for k in range(_J)]
    flat_base = [jax.lax.shift_left(et_idx[k], 10) + in_lo for k in range(_J)]

    def fire(slot, t):
        i0 = slot * _BLK
        pltpu.sync_copy(x_hbm.at[pl.ds(t * _BATCH + b0, _BLK)],
                        idx_v.at[pl.ds(i0, _BLK)])
        pltpu.async_copy(tok_hbm.at[idx_v.at[pl.ds(i0, _BLK)]],
                         rows_v.at[slot], gsems[slot])

    def wait_gather(slot):
        pltpu.make_async_copy(tok_hbm.at[pl.ds(0, _BLK)],
                              rows_v.at[slot], gsems[slot]).wait()

    def drain_out(slot, t):
        for et in range(_ET):
            pltpu.make_async_copy(slab_v.at[slot, pl.ds(et * _TILE, _TILE)],
                                  out_hbm.at[t, et * _NW + wid],
                                  osems[slot]).wait()

    def transpose_add(slot, t):
        pcols = [pos_v[pl.ds(t * _EMBED + k * _LANES, _LANES)]
                 for k in range(_J)]

        @plsc.parallel_loop(0, _BLK, unroll=2)
        def _(b):
            for k in range(_J):
                v = rows_v[slot, b, pl.ds(k * _LANES, _LANES)] + pcols[k]
                plsc.store_scatter(slab_v.at[slot], [flat_base[k] + b], v)

    def proc(slot, t, drain_pred):
        wait_gather(slot)

        @pl.when(drain_pred)
        def _():
            drain_out(slot, t - 2)

        transpose_add(slot, t)
        for et in range(_ET):
            pltpu.async_copy(slab_v.at[slot, pl.ds(et * _TILE, _TILE)],
                             out_hbm.at[t, et * _NW + wid], osems[slot])

    def round_body(r, carry):
        t0 = r * 2
        fire(0, t0)

        @pl.when(r >= 1)
        def _():
            proc(1, t0 - 1, r >= 2)

        fire(1, t0 + 1)
        proc(0, t0, r >= 1)
        return carry

    lax.fori_loop(0, _ROUNDS, round_body, 0)
    proc(1, _MAXLEN - 1, True)
    drain_out(0, _MAXLEN - 2)
    drain_out(1, _MAXLEN - 1)


def kernel(x, token_table, pos_table):
    xt1d = x.astype(jnp.int32).T.reshape(-1)
    tok128 = jnp.pad(token_table, ((0, 0), (0, _PADDED - _EMBED)))
    pos1d = pos_table.reshape(-1)
    mesh = plsc.VectorSubcoreMesh(core_axis_name="c", subcore_axis_name="s")
    out = pl.kernel(
        _tec_body,
        out_type=jax.ShapeDtypeStruct((_MAXLEN, _ET * _NW, _TILE), jnp.float32),
        mesh=mesh,
        compiler_params=pltpu.CompilerParams(use_tc_tiling_on_sc=False,
                                             needs_layout_passes=False),
        scratch_types=[
            pltpu.VMEM((_MAXLEN * _EMBED,), jnp.float32),
            pltpu.VMEM((2 * _BLK,), jnp.int32),
            pltpu.VMEM((2, _BLK, _PADDED), jnp.float32),
            pltpu.VMEM((2, _ET * _TILE), jnp.float32),
            pltpu.SemaphoreType.DMA,
            pltpu.SemaphoreType.DMA,
            pltpu.SemaphoreType.DMA,
            pltpu.SemaphoreType.DMA,
        ],
    )(xt1d, tok128, pos1d)
    out5 = out.reshape(_MAXLEN, _ET, _NW, 8, _BLK)
    return out5.transpose(2, 4, 0, 1, 3).reshape(_BATCH, _MAXLEN, _EMBED)


# speedup vs baseline: 1.1341x; 1.1341x over previous
import jax
import jax.numpy as jnp
from jax import lax
from jax.experimental import pallas as pl
from jax.experimental.pallas import tpu as pltpu
from jax.experimental.pallas import tpu_sc as plsc

_MAXLEN = 200
_EMBED = 64
_VOCAB = 100000
_BATCH = 4096
_LANES = 16
_PADDED = 128

_NC = 2
_NS = 16
_NW = _NC * _NS
_BLK = _BATCH // _NW
_ET = _EMBED // 8
_TILE = 8 * _BLK
_ROUNDS = _MAXLEN // 2
_J = _EMBED // _LANES


def _tec_body(x_hbm, tok_hbm, pos_hbm, out_hbm, pos_v, idx_v, rows_v, slab_v,
              g0, g1, o0, o1):
    c = lax.axis_index("c")
    s = lax.axis_index("s")
    wid = s * _NC + c
    gsems = (g0, g1)
    osems = (o0, o1)
    pltpu.sync_copy(pos_hbm, pos_v)
    pltpu.sync_copy(x_hbm.at[:, pl.ds(wid * _BLK, _BLK)], idx_v)
    lane = jax.lax.iota(jnp.int32, _LANES)
    et_hi = jax.lax.shift_right_logical(lane, 3)
    in_lo = jax.lax.shift_left(jnp.bitwise_and(lane, 7), 7)
    et_idx = [et_hi + (k * _LANES) // 8 for k in range(_J)]

    def fire(slot, t):
        pltpu.async_copy(tok_hbm.at[idx_v.at[t]], rows_v.at[slot],
                         gsems[slot])

    def wait_gather(slot):
        pltpu.make_async_copy(tok_hbm.at[pl.ds(0, _BLK)],
                              rows_v.at[slot], gsems[slot]).wait()

    def drain_out(slot, t):
        pltpu.make_async_copy(slab_v.at[slot], out_hbm.at[t, :, wid],
                              osems[slot]).wait()

    def transpose_add(slot, t):
        pcols = [pos_v[pl.ds(t * _EMBED + k * _LANES, _LANES)]
                 for k in range(_J)]

        @plsc.parallel_loop(0, _BLK, unroll=4)
        def _(b):
            for k in range(_J):
                v = rows_v[slot, b, pl.ds(k * _LANES, _LANES)] + pcols[k]
                plsc.store_scatter(slab_v.at[slot],
                                   [et_idx[k], in_lo + b], v)

    def proc(slot, t, drain_pred):
        wait_gather(slot)

        @pl.when(drain_pred)
        def _():
            drain_out(slot, t - 2)

        transpose_add(slot, t)
        pltpu.async_copy(slab_v.at[slot], out_hbm.at[t, :, wid], osems[slot])

    def round_body(r, carry):
        t0 = r * 2
        fire(0, t0)

        @pl.when(r >= 1)
        def _():
            proc(1, t0 - 1, r >= 2)

        fire(1, t0 + 1)
        proc(0, t0, r >= 1)
        return carry

    lax.fori_loop(0, _ROUNDS, round_body, 0)
    proc(1, _MAXLEN - 1, True)
    drain_out(0, _MAXLEN - 2)
    drain_out(1, _MAXLEN - 1)


def kernel(x, token_table, pos_table):
    xt = x.astype(jnp.int32).T
    tok128 = jnp.pad(token_table, ((0, 0), (0, _PADDED - _EMBED)))
    pos1d = pos_table.reshape(-1)
    mesh = plsc.VectorSubcoreMesh(core_axis_name="c", subcore_axis_name="s")
    out = pl.kernel(
        _tec_body,
        out_type=jax.ShapeDtypeStruct((_MAXLEN, _ET, _NW, _TILE), jnp.float32),
        mesh=mesh,
        compiler_params=pltpu.CompilerParams(use_tc_tiling_on_sc=False,
                                             needs_layout_passes=False),
        scratch_types=[
            pltpu.VMEM((_MAXLEN * _EMBED,), jnp.float32),
            pltpu.VMEM((_MAXLEN, _BLK), jnp.int32),
            pltpu.VMEM((2, _BLK, _PADDED), jnp.float32),
            pltpu.VMEM((2, _ET, _TILE), jnp.float32),
            pltpu.SemaphoreType.DMA,
            pltpu.SemaphoreType.DMA,
            pltpu.SemaphoreType.DMA,
            pltpu.SemaphoreType.DMA,
        ],
    )(xt, tok128, pos1d)
    out5 = out.reshape(_MAXLEN, _ET, _NW, 8, _BLK)
    return out5.transpose(2, 4, 0, 1, 3).reshape(_BATCH, _MAXLEN, _EMBED)


# speedup vs baseline: 1.3626x; 1.2015x over previous
import jax
import jax.numpy as jnp
from jax import lax
from jax.experimental import pallas as pl
from jax.experimental.pallas import tpu as pltpu
from jax.experimental.pallas import tpu_sc as plsc

_MAXLEN = 200
_EMBED = 64
_VOCAB = 100000
_BATCH = 4096
_LANES = 16
_PADDED = 128

_NC = 2
_NS = 16
_NW = _NC * _NS
_BPW = _BATCH // _NW
_ROUNDS = _BPW // 2
_J = _EMBED // _LANES


def _tec_body(x_hbm, tok_hbm, pos_hbm, out_hbm, pos_v, idx_v, rows_v, outs_v,
              g0, g1, o0, o1):
    c = lax.axis_index("c")
    s = lax.axis_index("s")
    wid = s * _NC + c
    gsems = (g0, g1)
    osems = (o0, o1)
    pltpu.sync_copy(pos_hbm, pos_v)
    elem_base = wid * _BPW

    def fire(slot, b):
        i0 = slot * _MAXLEN
        pltpu.sync_copy(x_hbm.at[pl.ds(b * _MAXLEN, _MAXLEN)],
                        idx_v.at[pl.ds(i0, _MAXLEN)])
        pltpu.async_copy(tok_hbm.at[idx_v.at[pl.ds(i0, 128)]],
                         rows_v.at[slot, pl.ds(0, 128)], gsems[slot])
        pltpu.async_copy(tok_hbm.at[idx_v.at[pl.ds(i0 + 128, 72)]],
                         rows_v.at[slot, pl.ds(128, 72)], gsems[slot])

    def wait_gathers(slot):
        pltpu.make_async_copy(tok_hbm.at[pl.ds(0, _MAXLEN)],
                              rows_v.at[slot], gsems[slot]).wait()

    def drain_out(slot, b):
        pltpu.make_async_copy(outs_v.at[slot], out_hbm.at[b],
                              osems[slot]).wait()

    def add_positions(slot):
        @plsc.parallel_loop(0, _MAXLEN, unroll=2)
        def _(t):
            for jj in range(_J):
                p = pos_v[pl.ds(t * _EMBED + jj * _LANES, _LANES)]
                outs_v[slot, t, pl.ds(jj * _LANES, _LANES)] = (
                    rows_v[slot, t, pl.ds(jj * _LANES, _LANES)] + p
                )

    def proc(slot, b, drain_pred):
        wait_gathers(slot)

        @pl.when(drain_pred)
        def _():
            drain_out(slot, b - 2)

        add_positions(slot)
        pltpu.async_copy(outs_v.at[slot], out_hbm.at[b], osems[slot])

    def round_body(r, carry):
        b0 = elem_base + r * 2
        fire(0, b0)

        @pl.when(r >= 1)
        def _():
            proc(1, b0 - 1, r >= 2)

        fire(1, b0 + 1)
        proc(0, b0, r >= 1)
        return carry

    lax.fori_loop(0, _ROUNDS, round_body, 0)
    last = elem_base + _BPW - 1
    proc(1, last, True)
    drain_out(0, last - 1)
    drain_out(1, last)


def kernel(x, token_table, pos_table):
    x1d = x.reshape(-1).astype(jnp.int32)
    tok128 = jnp.pad(token_table, ((0, 0), (0, _PADDED - _EMBED)))
    pos1d = pos_table.reshape(-1)
    mesh = plsc.VectorSubcoreMesh(core_axis_name="c", subcore_axis_name="s")
    out = pl.kernel(
        _tec_body,
        out_type=jax.ShapeDtypeStruct((_BATCH, _MAXLEN, _EMBED), jnp.float32),
        mesh=mesh,
        compiler_params=pltpu.CompilerParams(use_tc_tiling_on_sc=True),
        scratch_types=[
            pltpu.VMEM((_MAXLEN * _EMBED,), jnp.float32),
            pltpu.VMEM((2 * _MAXLEN,), jnp.int32),
            pltpu.VMEM((2, _MAXLEN, _PADDED), jnp.float32),
            pltpu.VMEM((2, _MAXLEN, _EMBED), jnp.float32),
            pltpu.SemaphoreType.DMA,
            pltpu.SemaphoreType.DMA,
            pltpu.SemaphoreType.DMA,
            pltpu.SemaphoreType.DMA,
        ],
    )(x1d, tok128, pos1d)
    return out


# speedup vs baseline: 1.3633x; 1.0005x over previous
import jax
import jax.numpy as jnp
from jax import lax
from jax.experimental import pallas as pl
from jax.experimental.pallas import tpu as pltpu
from jax.experimental.pallas import tpu_sc as plsc

_MAXLEN = 200
_EMBED = 64
_VOCAB = 100000
_BATCH = 4096
_LANES = 16
_PADDED = 128

_NC = 2
_NS = 16
_NW = _NC * _NS
_BPW = _BATCH // _NW
_ROUNDS = _BPW // 2
_J = _EMBED // _LANES


def _tec_body(x_hbm, tok_hbm, pos_hbm, out_hbm, pos_v, idx_v, rows_v, outs_v,
              g0, g1, o0, o1, i0, i1):
    c = lax.axis_index("c")
    s = lax.axis_index("s")
    wid = s * _NC + c
    gsems = (g0, g1)
    osems = (o0, o1)
    isems = (i0, i1)
    pltpu.sync_copy(pos_hbm, pos_v)
    elem_base = wid * _BPW

    def stage_idx(sem, j, b):
        pltpu.async_copy(x_hbm.at[pl.ds(b * _MAXLEN, _MAXLEN)],
                         idx_v.at[pl.ds(pl.multiple_of(j * _MAXLEN, 8),
                                        _MAXLEN)],
                         isems[sem])

    def fire(slot, j, b):
        i0 = pl.multiple_of(j * _MAXLEN, 8)
        pltpu.make_async_copy(x_hbm.at[pl.ds(b * _MAXLEN, _MAXLEN)],
                              idx_v.at[pl.ds(i0, _MAXLEN)],
                              isems[slot]).wait()
        pltpu.async_copy(tok_hbm.at[idx_v.at[pl.ds(i0, 128)]],
                         rows_v.at[slot, pl.ds(0, 128)], gsems[slot])
        pltpu.async_copy(tok_hbm.at[idx_v.at[pl.ds(i0 + 128, 72)]],
                         rows_v.at[slot, pl.ds(128, 72)], gsems[slot])

    def wait_gathers(slot):
        pltpu.make_async_copy(tok_hbm.at[pl.ds(0, _MAXLEN)],
                              rows_v.at[slot], gsems[slot]).wait()

    def drain_out(slot, b):
        pltpu.make_async_copy(outs_v.at[slot], out_hbm.at[b],
                              osems[slot]).wait()

    def add_positions(slot):
        @plsc.parallel_loop(0, _MAXLEN, unroll=4)
        def _(t):
            for jj in range(_J):
                p = pos_v[pl.ds(t * _EMBED + jj * _LANES, _LANES)]
                outs_v[slot, t, pl.ds(jj * _LANES, _LANES)] = (
                    rows_v[slot, t, pl.ds(jj * _LANES, _LANES)] + p
                )

    def proc(slot, b, drain_pred):
        wait_gathers(slot)

        @pl.when(drain_pred)
        def _():
            drain_out(slot, b - 2)

        add_positions(slot)
        pltpu.async_copy(outs_v.at[slot], out_hbm.at[b], osems[slot])

    def round_body(r, carry):
        b0 = elem_base + r * 2
        j0 = (r * 2) % 4
        fire(0, j0, b0)

        @pl.when(r < _ROUNDS - 1)
        def _():
            stage_idx(0, (j0 + 2) % 4, b0 + 2)

        @pl.when(r >= 1)
        def _():
            proc(1, b0 - 1, r >= 2)

        fire(1, j0 + 1, b0 + 1)

        @pl.when(r < _ROUNDS - 1)
        def _():
            stage_idx(1, (j0 + 3) % 4, b0 + 3)

        proc(0, b0, r >= 1)
        return carry

    stage_idx(0, 0, elem_base)
    stage_idx(1, 1, elem_base + 1)
    lax.fori_loop(0, _ROUNDS, round_body, 0)
    last = elem_base + _BPW - 1
    proc(1, last, True)
    drain_out(0, last - 1)
    drain_out(1, last)


def kernel(x, token_table, pos_table):
    x1d = x.reshape(-1).astype(jnp.int32)
    tok128 = jnp.pad(token_table, ((0, 0), (0, _PADDED - _EMBED)))
    pos1d = pos_table.reshape(-1)
    mesh = plsc.VectorSubcoreMesh(core_axis_name="c", subcore_axis_name="s")
    out = pl.kernel(
        _tec_body,
        out_type=jax.ShapeDtypeStruct((_BATCH, _MAXLEN, _EMBED), jnp.float32),
        mesh=mesh,
        compiler_params=pltpu.CompilerParams(use_tc_tiling_on_sc=True),
        scratch_types=[
            pltpu.VMEM((_MAXLEN * _EMBED,), jnp.float32),
            pltpu.VMEM((4 * _MAXLEN,), jnp.int32),
            pltpu.VMEM((2, _MAXLEN, _PADDED), jnp.float32),
            pltpu.VMEM((2, _MAXLEN, _EMBED), jnp.float32),
            pltpu.SemaphoreType.DMA,
            pltpu.SemaphoreType.DMA,
            pltpu.SemaphoreType.DMA,
            pltpu.SemaphoreType.DMA,
            pltpu.SemaphoreType.DMA,
            pltpu.SemaphoreType.DMA,
        ],
    )(x1d, tok128, pos1d)
    return out
